# baseline (speedup 1.0000x reference)
import functools

import jax
import jax.numpy as jnp
from jax import lax
from jax.experimental import pallas as pl
from jax.experimental.pallas import tpu as pltpu
from jax.experimental.pallas import tpu_sc as plsc

DIM = 64
PADW = 128
BATCH = 4096
HIST = 200
VOCAB = 1000000

NC = 2
NS = 16
NW = NC * NS
BBLK = BATCH // NW
L = 16

TBLK = 2048
NTBLK = (VOCAB + TBLK - 1) // TBLK


@functools.partial(
    pl.pallas_call,
    grid=(NTBLK,),
    in_specs=[pl.BlockSpec((DIM, TBLK), lambda j: (0, j))],
    out_specs=pl.BlockSpec((TBLK, PADW), lambda j: (j, 0)),
    out_shape=jax.ShapeDtypeStruct((VOCAB, PADW), jnp.float32),
)
def _transpose_pad(tT_ref, out_ref):
    t = tT_ref[...].T
    out_ref[...] = jnp.concatenate([t, t], axis=1)


@functools.partial(
    pl.kernel,
    mesh=plsc.VectorSubcoreMesh(core_axis_name="c", subcore_axis_name="s"),
    out_type=jax.ShapeDtypeStruct((HIST, DIM, BATCH), jnp.float32),
    scratch_types=[
        pltpu.VMEM((HIST, BBLK), jnp.int32),
        pltpu.VMEM((BBLK, PADW), jnp.float32),
        pltpu.VMEM((BBLK, PADW), jnp.float32),
        pltpu.VMEM((DIM, BBLK), jnp.float32),
        pltpu.VMEM((DIM, BBLK), jnp.float32),
        pltpu.SemaphoreType.DMA,
        pltpu.SemaphoreType.DMA,
        pltpu.SemaphoreType.DMA,
        pltpu.SemaphoreType.DMA,
        pltpu.SemaphoreType.DMA,
    ],
    compiler_params=pltpu.CompilerParams(
        skip_device_barrier=True, needs_layout_passes=False
    ),
)
def _emb_lookup(
    table_hbm, idxt_hbm, out_hbm,
    idx_v, stg0, stg1, ob0, ob1, gs0, gs1, ws0, ws1, isem,
):
    c = lax.axis_index("c")
    s = lax.axis_index("s")
    w = s * NC + c

    pltpu.async_copy(idxt_hbm.at[:, w], idx_v, isem).wait()

    stgs = (stg0, stg1)
    obs = (ob0, ob1)
    gsems = (gs0, gs1)
    wsems = (ws0, ws1)

    rows = [lax.iota(jnp.int32, L) + bsub * L for bsub in range(BBLK // L)]

    def fire_gather(h, p):
        pltpu.async_copy(table_hbm.at[idx_v.at[h]], stgs[p], gsems[p])

    def wait_gather(p):
        pltpu.make_async_copy(table_hbm.at[pl.ds(0, BBLK)], stgs[p], gsems[p]).wait()

    def fire_write(h, p):
        pltpu.async_copy(obs[p], out_hbm.at[h, :, pl.ds(w * BBLK, BBLK)], wsems[p])

    def wait_write(p):
        pltpu.make_async_copy(
            out_hbm.at[0, :, pl.ds(0, BBLK)], obs[p], wsems[p]
        ).wait()

    def transpose(p):
        stg = stgs[p]
        ob = obs[p]
        for d in range(DIM):
            colv = jnp.full((L,), d, dtype=jnp.int32)
            for bsub in range(BBLK // L):
                v = plsc.load_gather(stg, [rows[bsub], colv])
                ob[d, pl.ds(bsub * L, L)] = v

    fire_gather(0, 0)

    def hbody(hp, carry):
        h0 = hp * 2
        wait_gather(0)
        fire_gather(h0 + 1, 1)
        pl.when(hp > 0)(lambda: wait_write(0))
        transpose(0)
        fire_write(h0, 0)

        wait_gather(1)
        pl.when(h0 + 2 < HIST)(lambda: fire_gather(h0 + 2, 0))
        pl.when(hp > 0)(lambda: wait_write(1))
        transpose(1)
        fire_write(h0 + 1, 1)
        return carry

    lax.fori_loop(0, HIST // 2, hbody, None)
    wait_write(0)
    wait_write(1)


def kernel(inputs, emb_weight):
    table = _transpose_pad(emb_weight.T)
    idxt = inputs.T.reshape(HIST, NW, BBLK)
    out = _emb_lookup(table, idxt)
    return out.transpose(2, 0, 1)


# speedup vs baseline: 1.6460x; 1.6460x over previous
import functools

import jax
import jax.numpy as jnp
from jax import lax
from jax.experimental import pallas as pl
from jax.experimental.pallas import tpu as pltpu
from jax.experimental.pallas import tpu_sc as plsc

DIM = 64
PADW = 128
BATCH = 4096
HIST = 200
VOCAB = 1000000

NC = 2
NS = 16
NW = NC * NS
BBLK = BATCH // NW
L = 16

TBLK = 2048
NTBLK = (VOCAB + TBLK - 1) // TBLK


@functools.partial(
    pl.pallas_call,
    grid=(NTBLK,),
    in_specs=[pl.BlockSpec((DIM, TBLK), lambda j: (0, j))],
    out_specs=pl.BlockSpec((TBLK, PADW), lambda j: (j, 0)),
    out_shape=jax.ShapeDtypeStruct((VOCAB, PADW), jnp.float32),
)
def _transpose_pad(tT_ref, out_ref):
    t = tT_ref[...].T
    out_ref[...] = jnp.concatenate([t, t], axis=1)


@functools.partial(
    pl.kernel,
    mesh=plsc.VectorSubcoreMesh(core_axis_name="c", subcore_axis_name="s"),
    out_type=jax.ShapeDtypeStruct((HIST, DIM, BATCH), jnp.float32),
    scratch_types=[
        pltpu.VMEM((HIST, BBLK), jnp.int32),
        pltpu.VMEM((BBLK, PADW), jnp.float32),
        pltpu.VMEM((BBLK, PADW), jnp.float32),
        pltpu.VMEM((DIM, BBLK), jnp.float32),
        pltpu.VMEM((DIM, BBLK), jnp.float32),
        pltpu.SemaphoreType.DMA,
        pltpu.SemaphoreType.DMA,
        pltpu.SemaphoreType.DMA,
        pltpu.SemaphoreType.DMA,
        pltpu.SemaphoreType.DMA,
    ],
    compiler_params=pltpu.CompilerParams(
        skip_device_barrier=True, needs_layout_passes=False
    ),
)
def _emb_lookup(
    table_hbm, idxt_hbm, out_hbm,
    idx_v, stg0, stg1, ob0, ob1, gs0, gs1, ws0, ws1, isem,
):
    c = lax.axis_index("c")
    s = lax.axis_index("s")
    w = s * NC + c

    pltpu.async_copy(idxt_hbm.at[:, w], idx_v, isem).wait()

    stgs = (stg0, stg1)
    obs = (ob0, ob1)
    gsems = (gs0, gs1)
    wsems = (ws0, ws1)

    rows = [lax.iota(jnp.int32, L) + bsub * L for bsub in range(BBLK // L)]

    def fire_gather(h, p):
        pltpu.async_copy(table_hbm.at[idx_v.at[h]], stgs[p], gsems[p])

    def wait_gather(p):
        pltpu.make_async_copy(table_hbm.at[pl.ds(0, BBLK)], stgs[p], gsems[p]).wait()

    def fire_write(h, p):
        pltpu.async_copy(obs[p], out_hbm.at[h, :, pl.ds(w * BBLK, BBLK)], wsems[p])

    def wait_write(p):
        pltpu.make_async_copy(
            out_hbm.at[0, :, pl.ds(0, BBLK)], obs[p], wsems[p]
        ).wait()

    def transpose(p):
        stg = stgs[p]
        ob = obs[p]

        @plsc.parallel_loop(0, DIM, unroll=4)
        def dbody(d):
            colv = jnp.full((L,), d, dtype=jnp.int32)
            for bsub in range(BBLK // L):
                v = plsc.load_gather(stg, [rows[bsub], colv])
                ob[d, pl.ds(bsub * L, L)] = v

    fire_gather(0, 0)

    def hbody(hp, carry):
        h0 = hp * 2
        wait_gather(0)
        fire_gather(h0 + 1, 1)
        pl.when(hp > 0)(lambda: wait_write(0))
        transpose(0)
        fire_write(h0, 0)

        wait_gather(1)
        pl.when(h0 + 2 < HIST)(lambda: fire_gather(h0 + 2, 0))
        pl.when(hp > 0)(lambda: wait_write(1))
        transpose(1)
        fire_write(h0 + 1, 1)
        return carry

    lax.fori_loop(0, HIST // 2, hbody, None)
    wait_write(0)
    wait_write(1)


def kernel(inputs, emb_weight):
    table = _transpose_pad(emb_weight.T)
    idxt = inputs.T.reshape(HIST, NW, BBLK)
    out = _emb_lookup(table, idxt)
    return out.transpose(2, 0, 1)


# speedup vs baseline: 2.4452x; 1.4855x over previous
import functools

import jax
import jax.numpy as jnp
from jax import lax
from jax.experimental import pallas as pl
from jax.experimental.pallas import tpu as pltpu
from jax.experimental.pallas import tpu_sc as plsc

DIM = 64
PADW = 128
BATCH = 4096
HIST = 200
VOCAB = 1000000

NC = 2
NS = 16
NW = NC * NS
TOTAL = BATCH * HIST
PER_W = TOTAL // NW
G = 128
KSUB = 5
CHUNK = G * KSUB
NCHUNK = PER_W // CHUNK
NIDX = PER_W // G
L = 16

TBLK = 2048
NTBLK = (VOCAB + TBLK - 1) // TBLK
HBLK = TBLK // 2
VROWS = NTBLK * TBLK


@functools.partial(
    pl.pallas_call,
    grid=(NTBLK,),
    in_specs=[pl.BlockSpec((DIM, TBLK), lambda j: (0, j))],
    out_specs=pl.BlockSpec((HBLK, PADW), lambda j: (j, 0)),
    out_shape=jax.ShapeDtypeStruct((VROWS // 2, PADW), jnp.float32),
)
def _table_rows(tT_ref, out_ref):
    t = tT_ref[...].T
    out_ref[...] = jnp.concatenate([t[:HBLK], t[HBLK:]], axis=1)


@functools.partial(
    pl.kernel,
    mesh=plsc.VectorSubcoreMesh(core_axis_name="c", subcore_axis_name="s"),
    out_type=jax.ShapeDtypeStruct((2 * TOTAL, DIM), jnp.float32),
    scratch_types=[
        pltpu.VMEM((NIDX, G), jnp.int32),
        pltpu.VMEM((CHUNK, DIM), jnp.float32),
        pltpu.VMEM((CHUNK, DIM), jnp.float32),
        pltpu.VMEM((KSUB, G), jnp.int32),
        pltpu.VMEM((KSUB, G), jnp.int32),
        pltpu.SemaphoreType.DMA,
        pltpu.SemaphoreType.DMA,
        pltpu.SemaphoreType.DMA,
        pltpu.SemaphoreType.DMA,
    ],
    compiler_params=pltpu.CompilerParams(
        skip_device_barrier=True, use_tc_tiling_on_sc=False
    ),
)
def _emb_lookup(
    table_hbm, idx_hbm, out_hbm,
    idx_v, buf0, buf1, db0, db1, gs0, gs1, ss0, ss1,
):
    c = lax.axis_index("c")
    s = lax.axis_index("s")
    wid = s * NC + c
    base = wid * PER_W

    pltpu.sync_copy(idx_hbm.at[wid], idx_v)

    @plsc.parallel_loop(0, NIDX)
    def _remap(r):
        for sub in range(G // L):
            v = idx_v[r, pl.ds(sub * L, L)]
            g = (
                lax.shift_left(lax.shift_right_logical(v, 11), 11)
                + lax.shift_left(v & 1023, 1)
                + (lax.shift_right_logical(v, 10) & 1)
            )
            idx_v[r, pl.ds(sub * L, L)] = g

    bufs = (buf0, buf1)
    dbufs = (db0, db1)
    gsems = (gs0, gs1)
    ssems = (ss0, ss1)

    twoiota = 2 * lax.iota(jnp.int32, L)

    def fire_gather(i, b):
        for j in range(KSUB):
            pltpu.async_copy(
                table_hbm.at[idx_v.at[i * KSUB + j]],
                bufs[b].at[pl.ds(j * G, G)],
                gsems[b],
            )

    def wait_gather(b):
        pltpu.make_async_copy(table_hbm.at[pl.ds(0, CHUNK)], bufs[b], gsems[b]).wait()

    def fire_scatter(i, b):
        base2 = 2 * (base + i * CHUNK)
        for j in range(KSUB):
            for sub in range(G // L):
                off = base2 + 2 * (j * G + sub * L)
                dbufs[b][j, pl.ds(sub * L, L)] = twoiota + off
        for j in range(KSUB):
            pltpu.async_copy(
                bufs[b].at[pl.ds(j * G, G)],
                out_hbm.at[dbufs[b].at[j]],
                ssems[b],
            )

    def wait_scatter(b):
        pltpu.make_async_copy(
            out_hbm.at[pl.ds(0, CHUNK)], bufs[b], ssems[b]
        ).wait()

    fire_gather(0, 0)

    def outer(io, carry):
        i0 = io * 2
        wait_gather(0)
        fire_scatter(i0, 0)
        pl.when(i0 > 0)(lambda: wait_scatter(1))
        fire_gather(i0 + 1, 1)

        wait_gather(1)
        fire_scatter(i0 + 1, 1)
        wait_scatter(0)
        pl.when(i0 < NCHUNK - 2)(lambda: fire_gather(i0 + 2, 0))
        return carry

    lax.fori_loop(0, NCHUNK // 2, outer, None)
    wait_scatter(1)


def kernel(inputs, emb_weight):
    table = _table_rows(emb_weight.T).reshape(VROWS, DIM)
    idx = inputs.reshape(NW, NIDX, G)
    out = _emb_lookup(table, idx)
    return out.reshape(TOTAL, PADW)[:, :DIM].reshape(BATCH, HIST, DIM)


# speedup vs baseline: 3.1974x; 1.3076x over previous
import functools

import jax
import jax.numpy as jnp
from jax import lax
from jax.experimental import pallas as pl
from jax.experimental.pallas import tpu as pltpu
from jax.experimental.pallas import tpu_sc as plsc

DIM = 64
PADW = 128
BATCH = 4096
HIST = 200
VOCAB = 1000000

NC = 2
NS = 16
NW = NC * NS
TOTAL = BATCH * HIST
PER_W = TOTAL // NW
G = 128
KSUB = 5
CHUNK = G * KSUB
NCHUNK = PER_W // CHUNK
NIDX = PER_W // G
L = 16

TBLK = 8192
NTBLK = (VOCAB + TBLK - 1) // TBLK
HBLK = TBLK // 2
VROWS = NTBLK * TBLK


@functools.partial(
    pl.pallas_call,
    grid=(NTBLK,),
    in_specs=[pl.BlockSpec((DIM, TBLK), lambda j: (0, j))],
    out_specs=pl.BlockSpec((HBLK, PADW), lambda j: (j, 0)),
    out_shape=jax.ShapeDtypeStruct((VROWS // 2, PADW), jnp.float32),
)
def _table_rows(tT_ref, out_ref):
    t = tT_ref[...].T
    out_ref[...] = jnp.concatenate([t[:HBLK], t[HBLK:]], axis=1)


@functools.partial(
    pl.kernel,
    mesh=plsc.VectorSubcoreMesh(core_axis_name="c", subcore_axis_name="s"),
    out_type=jax.ShapeDtypeStruct((2 * TOTAL, DIM), jnp.float32),
    scratch_types=[
        pltpu.VMEM((NIDX, G), jnp.int32),
        pltpu.VMEM((CHUNK, DIM), jnp.float32),
        pltpu.VMEM((CHUNK, DIM), jnp.float32),
        pltpu.VMEM((KSUB, G), jnp.int32),
        pltpu.VMEM((KSUB, G), jnp.int32),
        pltpu.SemaphoreType.DMA,
        pltpu.SemaphoreType.DMA,
        pltpu.SemaphoreType.DMA,
        pltpu.SemaphoreType.DMA,
    ],
    compiler_params=pltpu.CompilerParams(
        skip_device_barrier=True, use_tc_tiling_on_sc=False
    ),
)
def _emb_lookup(
    table_hbm, idx_hbm, out_hbm,
    idx_v, buf0, buf1, db0, db1, gs0, gs1, ss0, ss1,
):
    c = lax.axis_index("c")
    s = lax.axis_index("s")
    wid = s * NC + c
    base = wid * PER_W

    pltpu.sync_copy(idx_hbm.at[wid], idx_v)

    sh = TBLK.bit_length() - 1
    @plsc.parallel_loop(0, NIDX)
    def _remap(r):
        for sub in range(G // L):
            v = idx_v[r, pl.ds(sub * L, L)]
            g = (
                lax.shift_left(lax.shift_right_logical(v, sh), sh)
                + lax.shift_left(v & (HBLK - 1), 1)
                + (lax.shift_right_logical(v, sh - 1) & 1)
            )
            idx_v[r, pl.ds(sub * L, L)] = g

    bufs = (buf0, buf1)
    dbufs = (db0, db1)
    gsems = (gs0, gs1)
    ssems = (ss0, ss1)

    twoiota = 2 * lax.iota(jnp.int32, L)

    def fire_gather(i, b):
        for j in range(KSUB):
            pltpu.async_copy(
                table_hbm.at[idx_v.at[i * KSUB + j]],
                bufs[b].at[pl.ds(j * G, G)],
                gsems[b],
            )

    def wait_gather(b):
        pltpu.make_async_copy(table_hbm.at[pl.ds(0, CHUNK)], bufs[b], gsems[b]).wait()

    def fire_scatter(i, b):
        base2 = 2 * (base + i * CHUNK)
        for j in range(KSUB):
            for sub in range(G // L):
                off = base2 + 2 * (j * G + sub * L)
                dbufs[b][j, pl.ds(sub * L, L)] = twoiota + off
        for j in range(KSUB):
            pltpu.async_copy(
                bufs[b].at[pl.ds(j * G, G)],
                out_hbm.at[dbufs[b].at[j]],
                ssems[b],
            )

    def wait_scatter(b):
        pltpu.make_async_copy(
            out_hbm.at[pl.ds(0, CHUNK)], bufs[b], ssems[b]
        ).wait()

    fire_gather(0, 0)

    def outer(io, carry):
        i0 = io * 2
        wait_gather(0)
        fire_scatter(i0, 0)
        pl.when(i0 > 0)(lambda: wait_scatter(1))
        fire_gather(i0 + 1, 1)

        wait_gather(1)
        fire_scatter(i0 + 1, 1)
        wait_scatter(0)
        pl.when(i0 < NCHUNK - 2)(lambda: fire_gather(i0 + 2, 0))
        return carry

    lax.fori_loop(0, NCHUNK // 2, outer, None)
    wait_scatter(1)


def kernel(inputs, emb_weight):
    table = _table_rows(emb_weight.T).reshape(VROWS, DIM)
    idx = inputs.reshape(NW, NIDX, G)
    out = _emb_lookup(table, idx)
    return out.reshape(TOTAL, PADW)[:, :DIM].reshape(BATCH, HIST, DIM)


# speedup vs baseline: 3.3785x; 1.0566x over previous
import functools

import jax
import jax.numpy as jnp
from jax import lax
from jax.experimental import pallas as pl
from jax.experimental.pallas import tpu as pltpu
from jax.experimental.pallas import tpu_sc as plsc

DIM = 64
PADW = 128
BATCH = 4096
HIST = 200
VOCAB = 1000000

NC = 2
NS = 16
NW = NC * NS
TOTAL = BATCH * HIST
PER_W = TOTAL // NW
G = 128
KSUB = 5
CHUNK = G * KSUB
NCHUNK = PER_W // CHUNK
NIDX = PER_W // G
L = 16

TBLK = 16384
NTBLK = (VOCAB + TBLK - 1) // TBLK
HBLK = TBLK // 2
VROWS = NTBLK * TBLK


@functools.partial(
    pl.pallas_call,
    grid=(NTBLK,),
    in_specs=[pl.BlockSpec((DIM, TBLK), lambda j: (0, j))],
    out_specs=pl.BlockSpec((HBLK, PADW), lambda j: (j, 0)),
    out_shape=jax.ShapeDtypeStruct((VROWS // 2, PADW), jnp.float32),
)
def _table_rows(tT_ref, out_ref):
    t = tT_ref[...].T
    out_ref[...] = jnp.concatenate([t[:HBLK], t[HBLK:]], axis=1)


@functools.partial(
    pl.kernel,
    mesh=plsc.VectorSubcoreMesh(core_axis_name="c", subcore_axis_name="s"),
    out_type=jax.ShapeDtypeStruct((2 * TOTAL, DIM), jnp.float32),
    scratch_types=[
        pltpu.VMEM((NIDX, G), jnp.int32),
        pltpu.VMEM((CHUNK, DIM), jnp.float32),
        pltpu.VMEM((CHUNK, DIM), jnp.float32),
        pltpu.VMEM((KSUB, G), jnp.int32),
        pltpu.VMEM((KSUB, G), jnp.int32),
        pltpu.SemaphoreType.DMA,
        pltpu.SemaphoreType.DMA,
        pltpu.SemaphoreType.DMA,
        pltpu.SemaphoreType.DMA,
    ],
    compiler_params=pltpu.CompilerParams(
        skip_device_barrier=True, use_tc_tiling_on_sc=False
    ),
)
def _emb_lookup(
    table_hbm, idx_hbm, out_hbm,
    idx_v, buf0, buf1, db0, db1, gs0, gs1, ss0, ss1,
):
    c = lax.axis_index("c")
    s = lax.axis_index("s")
    wid = s * NC + c
    base = wid * PER_W

    pltpu.sync_copy(idx_hbm.at[wid], idx_v)

    sh = TBLK.bit_length() - 1
    @plsc.parallel_loop(0, NIDX)
    def _remap(r):
        for sub in range(G // L):
            v = idx_v[r, pl.ds(sub * L, L)]
            g = (
                lax.shift_left(lax.shift_right_logical(v, sh), sh)
                + lax.shift_left(v & (HBLK - 1), 1)
                + (lax.shift_right_logical(v, sh - 1) & 1)
            )
            idx_v[r, pl.ds(sub * L, L)] = g

    bufs = (buf0, buf1)
    dbufs = (db0, db1)
    gsems = (gs0, gs1)
    ssems = (ss0, ss1)

    twoiota = 2 * lax.iota(jnp.int32, L)

    def fire_gather(i, b):
        for j in range(KSUB):
            pltpu.async_copy(
                table_hbm.at[idx_v.at[i * KSUB + j]],
                bufs[b].at[pl.ds(j * G, G)],
                gsems[b],
            )

    def wait_gather(b):
        pltpu.make_async_copy(table_hbm.at[pl.ds(0, CHUNK)], bufs[b], gsems[b]).wait()

    def fire_scatter(i, b):
        base2 = 2 * (base + i * CHUNK)
        for j in range(KSUB):
            for sub in range(G // L):
                off = base2 + 2 * (j * G + sub * L)
                dbufs[b][j, pl.ds(sub * L, L)] = twoiota + off
        for j in range(KSUB):
            pltpu.async_copy(
                bufs[b].at[pl.ds(j * G, G)],
                out_hbm.at[dbufs[b].at[j]],
                ssems[b],
            )

    def wait_scatter(b):
        pltpu.make_async_copy(
            out_hbm.at[pl.ds(0, CHUNK)], bufs[b], ssems[b]
        ).wait()

    fire_gather(0, 0)

    def outer(io, carry):
        i0 = io * 2
        wait_gather(0)
        fire_scatter(i0, 0)
        pl.when(i0 > 0)(lambda: wait_scatter(1))
        fire_gather(i0 + 1, 1)

        wait_gather(1)
        fire_scatter(i0 + 1, 1)
        wait_scatter(0)
        pl.when(i0 < NCHUNK - 2)(lambda: fire_gather(i0 + 2, 0))
        return carry

    lax.fori_loop(0, NCHUNK // 2, outer, None)
    wait_scatter(1)


def kernel(inputs, emb_weight):
    table = _table_rows(emb_weight.T).reshape(VROWS, DIM)
    idx = inputs.reshape(NW, NIDX, G)
    out = _emb_lookup(table, idx)
    return out.reshape(TOTAL, PADW)[:, :DIM].reshape(BATCH, HIST, DIM)


# speedup vs baseline: 3.4710x; 1.0274x over previous
import functools

import jax
import jax.numpy as jnp
from jax import lax
from jax.experimental import pallas as pl
from jax.experimental.pallas import tpu as pltpu
from jax.experimental.pallas import tpu_sc as plsc

DIM = 64
PADW = 128
BATCH = 4096
HIST = 200
VOCAB = 1000000

NC = 2
NS = 16
NW = NC * NS
TOTAL = BATCH * HIST
PER_W = TOTAL // NW
G = 128
KSUB = 5
CHUNK = G * KSUB
NCHUNK = PER_W // CHUNK
NIDX = PER_W // G
L = 16

TBLK = 32768
NTBLK = (VOCAB + TBLK - 1) // TBLK
HBLK = TBLK // 2
VROWS = NTBLK * TBLK


@functools.partial(
    pl.pallas_call,
    grid=(NTBLK,),
    in_specs=[pl.BlockSpec((DIM, TBLK), lambda j: (0, j))],
    out_specs=pl.BlockSpec((HBLK, PADW), lambda j: (j, 0)),
    out_shape=jax.ShapeDtypeStruct((VROWS // 2, PADW), jnp.float32),
)
def _table_rows(tT_ref, out_ref):
    t = tT_ref[...].T
    out_ref[...] = jnp.concatenate([t[:HBLK], t[HBLK:]], axis=1)


@functools.partial(
    pl.kernel,
    mesh=plsc.VectorSubcoreMesh(core_axis_name="c", subcore_axis_name="s"),
    out_type=jax.ShapeDtypeStruct((2 * TOTAL, DIM), jnp.float32),
    scratch_types=[
        pltpu.VMEM((NIDX, G), jnp.int32),
        pltpu.VMEM((CHUNK, DIM), jnp.float32),
        pltpu.VMEM((CHUNK, DIM), jnp.float32),
        pltpu.VMEM((KSUB, G), jnp.int32),
        pltpu.VMEM((KSUB, G), jnp.int32),
        pltpu.SemaphoreType.DMA,
        pltpu.SemaphoreType.DMA,
        pltpu.SemaphoreType.DMA,
        pltpu.SemaphoreType.DMA,
    ],
    compiler_params=pltpu.CompilerParams(
        skip_device_barrier=True, use_tc_tiling_on_sc=False
    ),
)
def _emb_lookup(
    table_hbm, idx_hbm, out_hbm,
    idx_v, buf0, buf1, db0, db1, gs0, gs1, ss0, ss1,
):
    c = lax.axis_index("c")
    s = lax.axis_index("s")
    wid = s * NC + c
    base = wid * PER_W

    pltpu.sync_copy(idx_hbm.at[wid], idx_v)

    sh = TBLK.bit_length() - 1
    @plsc.parallel_loop(0, NIDX)
    def _remap(r):
        for sub in range(G // L):
            v = idx_v[r, pl.ds(sub * L, L)]
            g = (
                lax.shift_left(lax.shift_right_logical(v, sh), sh)
                + lax.shift_left(v & (HBLK - 1), 1)
                + (lax.shift_right_logical(v, sh - 1) & 1)
            )
            idx_v[r, pl.ds(sub * L, L)] = g

    bufs = (buf0, buf1)
    dbufs = (db0, db1)
    gsems = (gs0, gs1)
    ssems = (ss0, ss1)

    twoiota = 2 * lax.iota(jnp.int32, L)

    def fire_gather(i, b):
        for j in range(KSUB):
            pltpu.async_copy(
                table_hbm.at[idx_v.at[i * KSUB + j]],
                bufs[b].at[pl.ds(j * G, G)],
                gsems[b],
            )

    def wait_gather(b):
        pltpu.make_async_copy(table_hbm.at[pl.ds(0, CHUNK)], bufs[b], gsems[b]).wait()

    def fire_scatter(i, b):
        base2 = 2 * (base + i * CHUNK)
        for j in range(KSUB):
            for sub in range(G // L):
                off = base2 + 2 * (j * G + sub * L)
                dbufs[b][j, pl.ds(sub * L, L)] = twoiota + off
        for j in range(KSUB):
            pltpu.async_copy(
                bufs[b].at[pl.ds(j * G, G)],
                out_hbm.at[dbufs[b].at[j]],
                ssems[b],
            )

    def wait_scatter(b):
        pltpu.make_async_copy(
            out_hbm.at[pl.ds(0, CHUNK)], bufs[b], ssems[b]
        ).wait()

    fire_gather(0, 0)

    def outer(io, carry):
        i0 = io * 2
        wait_gather(0)
        fire_scatter(i0, 0)
        pl.when(i0 > 0)(lambda: wait_scatter(1))
        fire_gather(i0 + 1, 1)

        wait_gather(1)
        fire_scatter(i0 + 1, 1)
        wait_scatter(0)
        pl.when(i0 < NCHUNK - 2)(lambda: fire_gather(i0 + 2, 0))
        return carry

    lax.fori_loop(0, NCHUNK // 2, outer, None)
    wait_scatter(1)


def kernel(inputs, emb_weight):
    table = _table_rows(emb_weight.T).reshape(VROWS, DIM)
    idx = inputs.reshape(NW, NIDX, G)
    out = _emb_lookup(table, idx)
    return out.reshape(TOTAL, PADW)[:, :DIM].reshape(BATCH, HIST, DIM)
